# Initial kernel scaffold; baseline (speedup 1.0000x reference)
import jax
import jax.numpy as jnp
from jax.experimental import pallas as pl


def kernel(u_hat, routing_num):
    raise NotImplementedError("write your pallas kernel here")


# speedup vs baseline: 13.2072x; 13.2072x over previous
import functools

import jax
import jax.numpy as jnp
from jax import lax
from jax.experimental import pallas as pl
from jax.experimental.pallas import tpu as pltpu
from jax.experimental.pallas import tpu_sc as plsc

_IN = 50000
_OUT = 32
_F = 16
_NW = 32
_CH = 64


def _make_pass():
    mesh = plsc.VectorSubcoreMesh(core_axis_name="c", subcore_axis_name="s")

    @functools.partial(
        pl.kernel,
        mesh=mesh,
        compiler_params=pltpu.CompilerParams(
            needs_layout_passes=False, use_tc_tiling_on_sc=False),
        out_type=jax.ShapeDtypeStruct((_NW, _OUT, _F), jnp.float32),
        scratch_types=[
            pltpu.VMEM((_CH * _OUT, _F), jnp.float32),
            pltpu.VMEM((_OUT, _F), jnp.float32),
            pltpu.VMEM((_OUT, _F), jnp.float32),
        ],
    )
    def sc_pass(u_hbm, vacc_hbm, out_hbm, ubuf, vaccv, sbuf):
        cid = lax.axis_index("c")
        sid = lax.axis_index("s")
        w = sid * 2 + cid
        start = (w * _IN) // _NW
        end = ((w + 1) * _IN) // _NW
        count = end - start
        nchunks = (count + _CH - 1) // _CH

        pltpu.sync_copy(vacc_hbm, vaccv)
        vrows = [vaccv[j, :] for j in range(_OUT)]
        for j in range(_OUT):
            sbuf[j, :] = jnp.zeros((_F,), jnp.float32)

        def chunk_body(k, carry):
            g = start + k * _CH
            d = jnp.minimum(g, end - _CH)
            lo = g - d
            pltpu.sync_copy(u_hbm.at[pl.ds(d * _OUT, _CH * _OUT)], ubuf)

            def node_body(n, c2):
                base = n * _OUT
                bs = [jnp.sum(ubuf[base + j, :] * vrows[j])
                      for j in range(_OUT)]
                ms = bs
                while len(ms) > 1:
                    ms = [jnp.maximum(ms[i], ms[i + 1])
                          for i in range(0, len(ms), 2)]
                m = ms[0]
                evs = [jnp.exp(jnp.full((_F,), bs[j] - m, jnp.float32))
                       for j in range(_OUT)]
                ts = evs
                while len(ts) > 1:
                    ts = [ts[i] + ts[i + 1] for i in range(0, len(ts), 2)]
                rv = 1.0 / ts[0]
                for j in range(_OUT):
                    u = ubuf[base + j, :]
                    plsc.addupdate(sbuf.at[j], u * (evs[j] * rv))
                return c2

            lax.fori_loop(lo, _CH, node_body, carry)
            return carry

        lax.fori_loop(0, nchunks, chunk_body, 0)
        pltpu.sync_copy(sbuf, out_hbm.at[w])

    return sc_pass


_sc_pass = _make_pass()


def _squash_v(s):
    sq = jnp.sum(s ** 2, axis=1, keepdims=True)
    return sq / (1.0 + sq) * (s / jnp.sqrt(sq))


def kernel(u_hat, routing_num):
    def body(_, carry):
        vacc, _v = carry
        parts = _sc_pass(u_hat, vacc)
        s = jnp.sum(parts, axis=0)
        v = _squash_v(s)
        return (vacc + v, v)

    init = (jnp.zeros((_OUT, _F), jnp.float32),
            jnp.zeros((_OUT, _F), jnp.float32))
    _, v = lax.fori_loop(0, routing_num, body, init)
    return v
